# Initial kernel scaffold; baseline (speedup 1.0000x reference)
import jax
import jax.numpy as jnp
from jax.experimental import pallas as pl


def kernel(h_v, edge_index, weight, bias):
    raise NotImplementedError("write your pallas kernel here")


# speedup vs baseline: 3.4044x; 3.4044x over previous
import functools

import jax
import jax.numpy as jnp
from jax import lax
from jax.experimental import pallas as pl
from jax.experimental.pallas import tpu as pltpu
from jax.experimental.pallas import tpu_sc as plsc

N_NODES = 10000
N_EDGES = 320000
F = 128

NC = 2
NS = 16
NW = NC * NS

CH = 128
EPT = 10240
NCHUNK = EPT // CH
E_PAD = EPT * NW
ACC_ROWS = 10240
ROWS_PER_TILE = ACC_ROWS // NS
DUMMY_DST = N_NODES


def _matmul_body(x_ref, w_ref, o_ref):
    o_ref[...] = jnp.dot(x_ref[...], w_ref[...],
                         preferred_element_type=jnp.float32)


def _combine_body(p0_ref, p1_ref, b_ref, o_ref):
    o_ref[...] = p0_ref[...] + p1_ref[...] + b_ref[...]


def _sc_scatter_kernel(support_hbm, src_hbm, dst_hbm, out_hbm,
                       src_idx_v, dst_idx_v, rows_v, acc_sh, sem):
    c = lax.axis_index("c")
    s = lax.axis_index("s")

    zero16 = jnp.zeros((16,), jnp.float32)

    def _zero_row(i, carry):
        for l in range(F // 16):
            rows_v[i, pl.ds(l * 16, 16)] = zero16
        return carry

    lax.fori_loop(0, CH, _zero_row, 0)
    for k in range(ROWS_PER_TILE // CH):
        pltpu.sync_copy(rows_v, acc_sh.at[pl.ds(s * ROWS_PER_TILE + k * CH, CH)])
    plsc.subcore_barrier()

    wid = c * NS + s
    pltpu.sync_copy(src_hbm.at[wid], src_idx_v)
    pltpu.sync_copy(dst_hbm.at[wid], dst_idx_v)

    def _chunk(j, carry):
        pltpu.async_copy(support_hbm.at[src_idx_v.at[j]], rows_v, sem).wait()
        pltpu.sync_copy(rows_v, acc_sh.at[dst_idx_v.at[j]], add=True)
        return carry

    lax.fori_loop(0, NCHUNK, _chunk, 0)

    plsc.subcore_barrier()
    pltpu.sync_copy(acc_sh.at[pl.ds(s * ROWS_PER_TILE, ROWS_PER_TILE)],
                    out_hbm.at[c, pl.ds(s * ROWS_PER_TILE, ROWS_PER_TILE)])


_sc_scatter = functools.partial(
    pl.kernel,
    out_type=jax.ShapeDtypeStruct((NC, ACC_ROWS, F), jnp.float32),
    mesh=plsc.VectorSubcoreMesh(core_axis_name="c", subcore_axis_name="s"),
    scratch_types=[
        pltpu.VMEM((NCHUNK, CH), jnp.int32),
        pltpu.VMEM((NCHUNK, CH), jnp.int32),
        pltpu.VMEM((CH, F), jnp.float32),
        pltpu.VMEM_SHARED((ACC_ROWS, F), jnp.float32),
        pltpu.SemaphoreType.DMA,
    ],
)(_sc_scatter_kernel)


def kernel(h_v, edge_index, weight, bias):
    rows_blk = 1000
    support = pl.pallas_call(
        _matmul_body,
        grid=(N_NODES // rows_blk,),
        in_specs=[
            pl.BlockSpec((rows_blk, F), lambda i: (i, 0)),
            pl.BlockSpec((F, F), lambda i: (0, 0)),
        ],
        out_specs=pl.BlockSpec((rows_blk, F), lambda i: (i, 0)),
        out_shape=jax.ShapeDtypeStruct((N_NODES, F), jnp.float32),
    )(h_v, weight)

    ei = edge_index.astype(jnp.int32)
    src = jnp.pad(ei[0], (0, E_PAD - N_EDGES)).reshape(NW, NCHUNK, CH)
    dst = jnp.pad(ei[1], (0, E_PAD - N_EDGES),
                  constant_values=DUMMY_DST).reshape(NW, NCHUNK, CH)

    partials = _sc_scatter(support, src, dst)

    out = pl.pallas_call(
        _combine_body,
        grid=(N_NODES // rows_blk,),
        in_specs=[
            pl.BlockSpec((rows_blk, F), lambda i: (i, 0)),
            pl.BlockSpec((rows_blk, F), lambda i: (i, 0)),
            pl.BlockSpec((1, F), lambda i: (0, 0)),
        ],
        out_specs=pl.BlockSpec((rows_blk, F), lambda i: (i, 0)),
        out_shape=jax.ShapeDtypeStruct((N_NODES, F), jnp.float32),
    )(partials[0, :N_NODES], partials[1, :N_NODES], bias.reshape(1, F))
    return out
